# baseline (speedup 1.0000x reference)
import jax
import jax.numpy as jnp
from jax import lax
from jax.experimental import pallas as pl
from jax.experimental.pallas import tpu as pltpu
from jax.experimental.pallas import tpu_sc as plsc

B = 16384
EMB = 64
HALF = 32
FEAT = 128
NC, NS = 2, 16
NW = NC * NS
BPW = B // NW
CH = 64
NCHUNK = BPW // CH
NROWTILES = 1000000 // 32


HB = BPW // 2


def _sc_gather_body(uidx_h, iidx_h, utab_h, itab_h,
                    ue_h, ie_h,
                    idx_v, rows, sem):
    wid = lax.axis_index("s") * NC + lax.axis_index("c")
    base = wid * BPW
    lanes = lax.iota(jnp.int32, 16)
    zeros = jnp.zeros((16,), jnp.int32)
    for tab_h, idx_h, emb_h in ((utab_h, uidx_h, ue_h), (itab_h, iidx_h, ie_h)):
        pltpu.sync_copy(idx_h.at[pl.ds(base, BPW)], idx_v)
        for h in range(2):
            def blk_body(k, _):
                raw = idx_v[pl.ds(h * HB + k * 16, 16)]
                for l in range(16):
                    ix = jnp.max(jnp.where(lanes == l, raw, zeros), axis=0)
                    v = lax.shift_left(lax.shift_right_logical(ix, 11), 10) \
                        + lax.bitwise_and(ix, 1023)
                    pltpu.async_copy(tab_h.at[v], rows.at[k * 16 + l], sem)
                return _
            lax.fori_loop(0, HB // 16, blk_body, 0)
            hsl = pl.ds(base + h * HB, HB)
            pltpu.make_async_copy(emb_h.at[hsl], rows, sem).wait()
            pltpu.sync_copy(rows, emb_h.at[hsl])


def _sc_gather(uidx, iidx, utab, itab):
    mesh = plsc.VectorSubcoreMesh(core_axis_name="c", subcore_axis_name="s")
    k = pl.kernel(
        _sc_gather_body,
        out_type=[
            jax.ShapeDtypeStruct((B, 2 * EMB), jnp.float32),
            jax.ShapeDtypeStruct((B, 2 * EMB), jnp.float32),
        ],
        mesh=mesh,
        scratch_types=[
            pltpu.VMEM((BPW,), jnp.int32),
            pltpu.VMEM((HB, 2 * EMB), jnp.float32),
            pltpu.SemaphoreType.DMA,
        ],
        compiler_params=pltpu.CompilerParams(use_tc_tiling_on_sc=True,
                                             needs_layout_passes=False),
    )
    return k(uidx, iidx, _tc_transpose(utab.T), _tc_transpose(itab.T))


BMC = 2048
NBLK = (1000000 + BMC - 1) // BMC


def _tc_transpose_body(x_r, o_r):
    xt = jnp.transpose(x_r[...])
    o_r[...] = jnp.concatenate([xt[: BMC // 2], xt[BMC // 2 :]], axis=-1)


def _tc_transpose(tabT):
    return pl.pallas_call(
        _tc_transpose_body,
        grid=(NBLK,),
        in_specs=[pl.BlockSpec((EMB, BMC), lambda i: (0, i))],
        out_specs=pl.BlockSpec((BMC // 2, 2 * EMB), lambda i: (i, 0)),
        out_shape=jax.ShapeDtypeStruct((NBLK * BMC // 2, 2 * EMB),
                                       jnp.float32),
    )(tabT)


BM = 2048


def _tc_combine_body(uf_r, wu_r, bu_r, if_r, wi_r, bi_r,
                     ue_r, ie_r, ct_r, st_r, ci_r, si_r, ui_r, ii_r,
                     uo_r, io_r):
    up = jnp.dot(uf_r[...], wu_r[...],
                 preferred_element_type=jnp.float32) + bu_r[...]
    ip = jnp.dot(if_r[...], wi_r[...],
                 preferred_element_type=jnp.float32) + bi_r[...]
    conehot = (ci_r[...] == lax.broadcasted_iota(jnp.int32, (1, 22), 1)
               ).astype(jnp.float32)
    sonehot = (si_r[...] == lax.broadcasted_iota(jnp.int32, (1, 18), 1)
               ).astype(jnp.float32)
    ce = jnp.dot(conehot, ct_r[...], preferred_element_type=jnp.float32)
    se = jnp.dot(sonehot, st_r[...], preferred_element_type=jnp.float32)
    uhi = lax.bitwise_and(lax.shift_right_logical(ui_r[...], 10), 1) == 1
    ihi = lax.bitwise_and(lax.shift_right_logical(ii_r[...], 10), 1) == 1
    ue = jnp.where(uhi, ue_r[:, EMB:], ue_r[:, :EMB])
    ie = jnp.where(ihi, ie_r[:, EMB:], ie_r[:, :EMB])
    uo_r[...] = jnp.concatenate([ue, ce, se, up], axis=-1)
    io_r[...] = jnp.concatenate([ie, ip], axis=-1)


def _tc_combine(uf, wu, bu, itf, wi, bi, ue, ie, ctab, stab, cidx, sidx,
                uidx, iidx):
    return pl.pallas_call(
        _tc_combine_body,
        grid=(B // BM,),
        in_specs=[
            pl.BlockSpec((BM, FEAT), lambda i: (i, 0)),
            pl.BlockSpec((FEAT, EMB), lambda i: (0, 0)),
            pl.BlockSpec((1, EMB), lambda i: (0, 0)),
            pl.BlockSpec((BM, FEAT), lambda i: (i, 0)),
            pl.BlockSpec((FEAT, EMB), lambda i: (0, 0)),
            pl.BlockSpec((1, EMB), lambda i: (0, 0)),
            pl.BlockSpec((BM, 2 * EMB), lambda i: (i, 0)),
            pl.BlockSpec((BM, 2 * EMB), lambda i: (i, 0)),
            pl.BlockSpec((22, HALF), lambda i: (0, 0)),
            pl.BlockSpec((18, HALF), lambda i: (0, 0)),
            pl.BlockSpec((BM, 1), lambda i: (i, 0)),
            pl.BlockSpec((BM, 1), lambda i: (i, 0)),
            pl.BlockSpec((BM, 1), lambda i: (i, 0)),
            pl.BlockSpec((BM, 1), lambda i: (i, 0)),
        ],
        out_specs=[
            pl.BlockSpec((BM, 3 * EMB), lambda i: (i, 0)),
            pl.BlockSpec((BM, 2 * EMB), lambda i: (i, 0)),
        ],
        out_shape=[
            jax.ShapeDtypeStruct((B, 3 * EMB), jnp.float32),
            jax.ShapeDtypeStruct((B, 2 * EMB), jnp.float32),
        ],
    )(uf, wu, bu, itf, wi, bi, ue, ie, ctab, stab, cidx, sidx, uidx, iidx)


def kernel(user_idx, user_features, user_color_idx, user_size_idx,
           item_idx, item_features, user_table, item_table,
           color_table, size_table, W_user, b_user, W_item, b_item):
    ue, ie = _sc_gather(user_idx, item_idx, user_table, item_table)
    uo, io = _tc_combine(user_features, W_user, b_user.reshape(1, EMB),
                         item_features, W_item, b_item.reshape(1, EMB),
                         ue, ie, color_table, size_table,
                         user_color_idx.reshape(B, 1),
                         user_size_idx.reshape(B, 1),
                         user_idx.reshape(B, 1), item_idx.reshape(B, 1))
    return uo, io


# speedup vs baseline: 2.0374x; 2.0374x over previous
import jax
import jax.numpy as jnp
from jax import lax
from jax.experimental import pallas as pl
from jax.experimental.pallas import tpu as pltpu
from jax.experimental.pallas import tpu_sc as plsc

B = 16384
EMB = 64
HALF = 32
FEAT = 128
NC, NS = 2, 16
NW = NC * NS
BPW = B // NW
HB = BPW // 2


def _sc_gather_body(uidx_h, iidx_h, utab_h, itab_h,
                    ue_h, ie_h,
                    idx_v, rows, sem):
    wid = lax.axis_index("s") * NC + lax.axis_index("c")
    base = wid * BPW
    lanes = lax.iota(jnp.int32, 16)
    zeros = jnp.zeros((16,), jnp.int32)
    for tab_h, idx_h, emb_h in ((utab_h, uidx_h, ue_h), (itab_h, iidx_h, ie_h)):
        pltpu.sync_copy(idx_h.at[pl.ds(base, BPW)], idx_v)
        for h in range(2):
            def blk_body(k, _):
                raw = idx_v[pl.ds(h * HB + k * 16, 16)]
                for l in range(16):
                    ix = jnp.max(jnp.where(lanes == l, raw, zeros), axis=0)
                    pltpu.async_copy(
                        tab_h.at[lax.shift_right_logical(ix, 5),
                                 lax.bitwise_and(ix, 31)],
                        rows.at[k * 16 + l], sem)
                return _
            lax.fori_loop(0, HB // 16, blk_body, 0)
            hsl = pl.ds(base + h * HB, HB)
            pltpu.make_async_copy(emb_h.at[hsl], rows, sem).wait()
            pltpu.sync_copy(rows, emb_h.at[hsl])


def _sc_gather(uidx, iidx, utab, itab):
    mesh = plsc.VectorSubcoreMesh(core_axis_name="c", subcore_axis_name="s")
    k = pl.kernel(
        _sc_gather_body,
        out_type=[
            jax.ShapeDtypeStruct((B, EMB), jnp.float32),
            jax.ShapeDtypeStruct((B, EMB), jnp.float32),
        ],
        mesh=mesh,
        scratch_types=[
            pltpu.VMEM((BPW,), jnp.int32),
            pltpu.VMEM((HB, EMB), jnp.float32),
            pltpu.SemaphoreType.DMA,
        ],
        compiler_params=pltpu.CompilerParams(use_tc_tiling_on_sc=True,
                                             needs_layout_passes=False),
    )
    return k(uidx, iidx, utab.reshape(31250, 32, EMB),
             itab.reshape(31250, 32, EMB))


BM = 2048


def _tc_combine_body(uf_r, wu_r, bu_r, if_r, wi_r, bi_r,
                     ue_r, ie_r, ct_r, st_r, ci_r, si_r, uo_r, io_r):
    ip = jnp.dot(if_r[...], wi_r[...],
                 preferred_element_type=jnp.float32) + bi_r[...]
    io_r[...] = jnp.concatenate([ie_r[...], ip], axis=-1)
    upt = jnp.dot(jnp.transpose(wu_r[...]), jnp.transpose(uf_r[...]),
                  preferred_element_type=jnp.float32) + bu_r[...]
    conehot = (ci_r[...] == lax.broadcasted_iota(jnp.int32, (22, 1), 0)
               ).astype(jnp.float32)
    sonehot = (si_r[...] == lax.broadcasted_iota(jnp.int32, (18, 1), 0)
               ).astype(jnp.float32)
    cet = jnp.dot(jnp.transpose(ct_r[...]), conehot,
                  preferred_element_type=jnp.float32)
    set_ = jnp.dot(jnp.transpose(st_r[...]), sonehot,
                   preferred_element_type=jnp.float32)
    uo_r[...] = jnp.concatenate(
        [jnp.transpose(ue_r[...]), cet, set_, upt], axis=0)


def _tc_combine(uf, wu, bu, itf, wi, bi, ue, ie, ctab, stab, cidx, sidx):
    return pl.pallas_call(
        _tc_combine_body,
        grid=(B // BM,),
        in_specs=[
            pl.BlockSpec((BM, FEAT), lambda i: (i, 0)),
            pl.BlockSpec((FEAT, EMB), lambda i: (0, 0)),
            pl.BlockSpec((EMB, 1), lambda i: (0, 0)),
            pl.BlockSpec((BM, FEAT), lambda i: (i, 0)),
            pl.BlockSpec((FEAT, EMB), lambda i: (0, 0)),
            pl.BlockSpec((1, EMB), lambda i: (0, 0)),
            pl.BlockSpec((BM, EMB), lambda i: (i, 0)),
            pl.BlockSpec((BM, EMB), lambda i: (i, 0)),
            pl.BlockSpec((22, HALF), lambda i: (0, 0)),
            pl.BlockSpec((18, HALF), lambda i: (0, 0)),
            pl.BlockSpec((1, BM), lambda i: (0, i)),
            pl.BlockSpec((1, BM), lambda i: (0, i)),
        ],
        out_specs=[
            pl.BlockSpec((3 * EMB, BM), lambda i: (0, i)),
            pl.BlockSpec((BM, 2 * EMB), lambda i: (i, 0)),
        ],
        out_shape=[
            jax.ShapeDtypeStruct((3 * EMB, B), jnp.float32),
            jax.ShapeDtypeStruct((B, 2 * EMB), jnp.float32),
        ],
    )(uf, wu, bu, itf, wi, bi, ue, ie, ctab, stab, cidx, sidx)


def kernel(user_idx, user_features, user_color_idx, user_size_idx,
           item_idx, item_features, user_table, item_table,
           color_table, size_table, W_user, b_user, W_item, b_item):
    ue, ie = _sc_gather(user_idx, item_idx, user_table, item_table)
    uot, io = _tc_combine(user_features, W_user, b_user.reshape(EMB, 1),
                          item_features, W_item, b_item.reshape(1, EMB),
                          ue, ie, color_table, size_table,
                          user_color_idx.reshape(1, B),
                          user_size_idx.reshape(1, B))
    return uot.T, io


# speedup vs baseline: 2.4510x; 1.2030x over previous
import jax
import jax.numpy as jnp
from jax import lax
from jax.experimental import pallas as pl
from jax.experimental.pallas import tpu as pltpu
from jax.experimental.pallas import tpu_sc as plsc

B = 16384
EMB = 64
HALF = 32
FEAT = 128
NC, NS = 2, 16
NW = NC * NS
BPW = B // NW
HB = BPW // 2


BMC = 8192
NBLK = (1000000 + BMC - 1) // BMC


def _tc_transpose_body(x_r, o_r):
    xt = jnp.transpose(x_r[...])
    o_r[...] = jnp.concatenate([xt[: BMC // 2], xt[BMC // 2 :]], axis=-1)


def _tc_transpose(tabT):
    return pl.pallas_call(
        _tc_transpose_body,
        grid=(NBLK,),
        in_specs=[pl.BlockSpec((EMB, BMC), lambda i: (0, i))],
        out_specs=pl.BlockSpec((BMC // 2, 2 * EMB), lambda i: (i, 0)),
        out_shape=jax.ShapeDtypeStruct((NBLK * BMC // 2, 2 * EMB),
                                       jnp.float32),
    )(tabT)


def _sc_gather_body(uidx_h, iidx_h, utab_h, itab_h,
                    ue_h, ie_h,
                    idx_v, urows, irows, sem):
    wid = lax.axis_index("s") * NC + lax.axis_index("c")
    base = wid * BPW
    lanes = lax.iota(jnp.int32, 16)
    zeros = jnp.zeros((16,), jnp.int32)
    pltpu.sync_copy(uidx_h.at[pl.ds(base, BPW)], idx_v)
    for h in range(2):
        def ublk(k, _):
            raw = idx_v[pl.ds(h * HB + k * 16, 16)]
            for l in range(16):
                ix = jnp.max(jnp.where(lanes == l, raw, zeros), axis=0)
                v = lax.shift_left(lax.shift_right_logical(ix, 13), 12) \
                    + lax.bitwise_and(ix, 4095)
                pltpu.async_copy(utab_h.at[v], urows.at[k * 16 + l], sem)
            return _
        lax.fori_loop(0, HB // 16, ublk, 0)
        hsl = pl.ds(base + h * HB, HB)
        pltpu.make_async_copy(ue_h.at[hsl], urows, sem).wait()
        pltpu.sync_copy(urows, ue_h.at[hsl])
    pltpu.sync_copy(iidx_h.at[pl.ds(base, BPW)], idx_v)
    for h in range(2):
        def iblk(k, _):
            raw = idx_v[pl.ds(h * HB + k * 16, 16)]
            for l in range(16):
                ix = jnp.max(jnp.where(lanes == l, raw, zeros), axis=0)
                pltpu.async_copy(
                    itab_h.at[lax.shift_right_logical(ix, 5),
                              lax.bitwise_and(ix, 31)],
                    irows.at[k * 16 + l], sem)
            return _
        lax.fori_loop(0, HB // 16, iblk, 0)
        hsl = pl.ds(base + h * HB, HB)
        pltpu.make_async_copy(ie_h.at[hsl], irows, sem).wait()
        pltpu.sync_copy(irows, ie_h.at[hsl])


def _sc_gather(uidx, iidx, utab_c, itab):
    mesh = plsc.VectorSubcoreMesh(core_axis_name="c", subcore_axis_name="s")
    k = pl.kernel(
        _sc_gather_body,
        out_type=[
            jax.ShapeDtypeStruct((B, 2 * EMB), jnp.float32),
            jax.ShapeDtypeStruct((B, EMB), jnp.float32),
        ],
        mesh=mesh,
        scratch_types=[
            pltpu.VMEM((BPW,), jnp.int32),
            pltpu.VMEM((HB, 2 * EMB), jnp.float32),
            pltpu.VMEM((HB, EMB), jnp.float32),
            pltpu.SemaphoreType.DMA,
        ],
        compiler_params=pltpu.CompilerParams(use_tc_tiling_on_sc=True,
                                             needs_layout_passes=False),
    )
    return k(uidx, iidx, utab_c, itab.reshape(31250, 32, EMB))


BM = 2048


def _tc_combine_body(uf_r, wu_r, bu_r, if_r, wi_r, bi_r,
                     ue_r, ie_r, ct_r, st_r, ci_r, si_r, ui_r, uo_r, io_r):
    ip = jnp.dot(if_r[...], wi_r[...],
                 preferred_element_type=jnp.float32) + bi_r[...]
    io_r[...] = jnp.concatenate([ie_r[...], ip], axis=-1)
    upt = jnp.dot(jnp.transpose(wu_r[...]), jnp.transpose(uf_r[...]),
                  preferred_element_type=jnp.float32) + bu_r[...]
    conehot = (ci_r[...] == lax.broadcasted_iota(jnp.int32, (22, 1), 0)
               ).astype(jnp.float32)
    sonehot = (si_r[...] == lax.broadcasted_iota(jnp.int32, (18, 1), 0)
               ).astype(jnp.float32)
    cet = jnp.dot(jnp.transpose(ct_r[...]), conehot,
                  preferred_element_type=jnp.float32)
    set_ = jnp.dot(jnp.transpose(st_r[...]), sonehot,
                   preferred_element_type=jnp.float32)
    uhi = lax.bitwise_and(lax.shift_right_logical(ui_r[...], 12), 1) == 1
    ue = jnp.where(uhi, ue_r[:, EMB:], ue_r[:, :EMB])
    uo_r[...] = jnp.concatenate([jnp.transpose(ue), cet, set_, upt], axis=0)


def _tc_combine(uf, wu, bu, itf, wi, bi, ue, ie, ctab, stab, cidx, sidx,
                uidx):
    return pl.pallas_call(
        _tc_combine_body,
        grid=(B // BM,),
        in_specs=[
            pl.BlockSpec((BM, FEAT), lambda i: (i, 0)),
            pl.BlockSpec((FEAT, EMB), lambda i: (0, 0)),
            pl.BlockSpec((EMB, 1), lambda i: (0, 0)),
            pl.BlockSpec((BM, FEAT), lambda i: (i, 0)),
            pl.BlockSpec((FEAT, EMB), lambda i: (0, 0)),
            pl.BlockSpec((1, EMB), lambda i: (0, 0)),
            pl.BlockSpec((BM, 2 * EMB), lambda i: (i, 0)),
            pl.BlockSpec((BM, EMB), lambda i: (i, 0)),
            pl.BlockSpec((22, HALF), lambda i: (0, 0)),
            pl.BlockSpec((18, HALF), lambda i: (0, 0)),
            pl.BlockSpec((1, BM), lambda i: (0, i)),
            pl.BlockSpec((1, BM), lambda i: (0, i)),
            pl.BlockSpec((BM, 1), lambda i: (i, 0)),
        ],
        out_specs=[
            pl.BlockSpec((3 * EMB, BM), lambda i: (0, i)),
            pl.BlockSpec((BM, 2 * EMB), lambda i: (i, 0)),
        ],
        out_shape=[
            jax.ShapeDtypeStruct((3 * EMB, B), jnp.float32),
            jax.ShapeDtypeStruct((B, 2 * EMB), jnp.float32),
        ],
    )(uf, wu, bu, itf, wi, bi, ue, ie, ctab, stab, cidx, sidx, uidx)


def kernel(user_idx, user_features, user_color_idx, user_size_idx,
           item_idx, item_features, user_table, item_table,
           color_table, size_table, W_user, b_user, W_item, b_item):
    utab_c = _tc_transpose(user_table.T)
    ue, ie = _sc_gather(user_idx, item_idx, utab_c, item_table)
    uot, io = _tc_combine(user_features, W_user, b_user.reshape(EMB, 1),
                          item_features, W_item, b_item.reshape(1, EMB),
                          ue, ie, color_table, size_table,
                          user_color_idx.reshape(1, B),
                          user_size_idx.reshape(1, B),
                          user_idx.reshape(B, 1))
    return uot.T, io


# speedup vs baseline: 2.4762x; 1.0103x over previous
import jax
import jax.numpy as jnp
from jax import lax
from jax.experimental import pallas as pl
from jax.experimental.pallas import tpu as pltpu
from jax.experimental.pallas import tpu_sc as plsc

B = 16384
EMB = 64
HALF = 32
FEAT = 128
NC, NS = 2, 16
NW = NC * NS
BPW = B // NW
HB = BPW // 2


BMC = 16384
NBLK = (1000000 + BMC - 1) // BMC
SH = BMC.bit_length() - 1


def _tc_transpose_body(x_r, o_r):
    xt = jnp.transpose(x_r[...])
    o_r[...] = jnp.concatenate([xt[: BMC // 2], xt[BMC // 2 :]], axis=-1)


def _tc_transpose(tabT):
    return pl.pallas_call(
        _tc_transpose_body,
        grid=(NBLK,),
        in_specs=[pl.BlockSpec((EMB, BMC), lambda i: (0, i))],
        out_specs=pl.BlockSpec((BMC // 2, 2 * EMB), lambda i: (i, 0)),
        out_shape=jax.ShapeDtypeStruct((NBLK * BMC // 2, 2 * EMB),
                                       jnp.float32),
    )(tabT)


def _sc_gather_body(uidx_h, iidx_h, utab_h, itab_h,
                    ue_h, ie_h,
                    idx_v, urows, irows, sem):
    wid = lax.axis_index("s") * NC + lax.axis_index("c")
    base = wid * BPW
    lanes = lax.iota(jnp.int32, 16)
    zeros = jnp.zeros((16,), jnp.int32)
    pltpu.sync_copy(uidx_h.at[pl.ds(base, BPW)], idx_v)
    for h in range(2):
        def ublk(k, _):
            raw = idx_v[pl.ds(h * HB + k * 16, 16)]
            for l in range(16):
                ix = jnp.max(jnp.where(lanes == l, raw, zeros), axis=0)
                v = lax.shift_left(lax.shift_right_logical(ix, SH), SH - 1) \
                    + lax.bitwise_and(ix, BMC // 2 - 1)
                pltpu.async_copy(utab_h.at[v], urows.at[k * 16 + l], sem)
            return _
        lax.fori_loop(0, HB // 16, ublk, 0)
        hsl = pl.ds(base + h * HB, HB)
        pltpu.make_async_copy(ue_h.at[hsl], urows, sem).wait()
        pltpu.sync_copy(urows, ue_h.at[hsl])
    pltpu.sync_copy(iidx_h.at[pl.ds(base, BPW)], idx_v)
    for h in range(2):
        def iblk(k, _):
            raw = idx_v[pl.ds(h * HB + k * 16, 16)]
            for l in range(16):
                ix = jnp.max(jnp.where(lanes == l, raw, zeros), axis=0)
                pltpu.async_copy(
                    itab_h.at[lax.shift_right_logical(ix, 5),
                              lax.bitwise_and(ix, 31)],
                    irows.at[k * 16 + l], sem)
            return _
        lax.fori_loop(0, HB // 16, iblk, 0)
        hsl = pl.ds(base + h * HB, HB)
        pltpu.make_async_copy(ie_h.at[hsl], irows, sem).wait()
        pltpu.sync_copy(irows, ie_h.at[hsl])


def _sc_gather(uidx, iidx, utab_c, itab):
    mesh = plsc.VectorSubcoreMesh(core_axis_name="c", subcore_axis_name="s")
    k = pl.kernel(
        _sc_gather_body,
        out_type=[
            jax.ShapeDtypeStruct((B, 2 * EMB), jnp.float32),
            jax.ShapeDtypeStruct((B, EMB), jnp.float32),
        ],
        mesh=mesh,
        scratch_types=[
            pltpu.VMEM((BPW,), jnp.int32),
            pltpu.VMEM((HB, 2 * EMB), jnp.float32),
            pltpu.VMEM((HB, EMB), jnp.float32),
            pltpu.SemaphoreType.DMA,
        ],
        compiler_params=pltpu.CompilerParams(use_tc_tiling_on_sc=True,
                                             needs_layout_passes=False),
    )
    return k(uidx, iidx, utab_c, itab.reshape(31250, 32, EMB))


BM = 2048


def _tc_combine_body(uf_r, wu_r, bu_r, if_r, wi_r, bi_r,
                     ue_r, ie_r, ct_r, st_r, ci_r, si_r, ui_r, uo_r, io_r):
    ip = jnp.dot(if_r[...], wi_r[...],
                 preferred_element_type=jnp.float32) + bi_r[...]
    io_r[...] = jnp.concatenate([ie_r[...], ip], axis=-1)
    upt = jnp.dot(jnp.transpose(wu_r[...]), jnp.transpose(uf_r[...]),
                  preferred_element_type=jnp.float32) + bu_r[...]
    conehot = (ci_r[...] == lax.broadcasted_iota(jnp.int32, (22, 1), 0)
               ).astype(jnp.float32)
    sonehot = (si_r[...] == lax.broadcasted_iota(jnp.int32, (18, 1), 0)
               ).astype(jnp.float32)
    cet = jnp.dot(jnp.transpose(ct_r[...]), conehot,
                  preferred_element_type=jnp.float32)
    set_ = jnp.dot(jnp.transpose(st_r[...]), sonehot,
                   preferred_element_type=jnp.float32)
    uhi = lax.bitwise_and(lax.shift_right_logical(ui_r[...], SH - 1), 1) == 1
    ue = jnp.where(uhi, ue_r[:, EMB:], ue_r[:, :EMB])
    uo_r[...] = jnp.concatenate([jnp.transpose(ue), cet, set_, upt], axis=0)


def _tc_combine(uf, wu, bu, itf, wi, bi, ue, ie, ctab, stab, cidx, sidx,
                uidx):
    return pl.pallas_call(
        _tc_combine_body,
        grid=(B // BM,),
        in_specs=[
            pl.BlockSpec((BM, FEAT), lambda i: (i, 0)),
            pl.BlockSpec((FEAT, EMB), lambda i: (0, 0)),
            pl.BlockSpec((EMB, 1), lambda i: (0, 0)),
            pl.BlockSpec((BM, FEAT), lambda i: (i, 0)),
            pl.BlockSpec((FEAT, EMB), lambda i: (0, 0)),
            pl.BlockSpec((1, EMB), lambda i: (0, 0)),
            pl.BlockSpec((BM, 2 * EMB), lambda i: (i, 0)),
            pl.BlockSpec((BM, EMB), lambda i: (i, 0)),
            pl.BlockSpec((22, HALF), lambda i: (0, 0)),
            pl.BlockSpec((18, HALF), lambda i: (0, 0)),
            pl.BlockSpec((1, BM), lambda i: (0, i)),
            pl.BlockSpec((1, BM), lambda i: (0, i)),
            pl.BlockSpec((BM, 1), lambda i: (i, 0)),
        ],
        out_specs=[
            pl.BlockSpec((3 * EMB, BM), lambda i: (0, i)),
            pl.BlockSpec((BM, 2 * EMB), lambda i: (i, 0)),
        ],
        out_shape=[
            jax.ShapeDtypeStruct((3 * EMB, B), jnp.float32),
            jax.ShapeDtypeStruct((B, 2 * EMB), jnp.float32),
        ],
    )(uf, wu, bu, itf, wi, bi, ue, ie, ctab, stab, cidx, sidx, uidx)


def kernel(user_idx, user_features, user_color_idx, user_size_idx,
           item_idx, item_features, user_table, item_table,
           color_table, size_table, W_user, b_user, W_item, b_item):
    utab_c = _tc_transpose(user_table.T)
    ue, ie = _sc_gather(user_idx, item_idx, utab_c, item_table)
    uot, io = _tc_combine(user_features, W_user, b_user.reshape(EMB, 1),
                          item_features, W_item, b_item.reshape(1, EMB),
                          ue, ie, color_table, size_table,
                          user_color_idx.reshape(1, B),
                          user_size_idx.reshape(1, B),
                          user_idx.reshape(B, 1))
    return uot.T, io
